# baseline (speedup 1.0000x reference)
import functools

import jax
import jax.numpy as jnp
from jax import lax
from jax.experimental import pallas as pl
from jax.experimental.pallas import tpu as pltpu
from jax.experimental.pallas import tpu_sc as plsc

N = 10000
E = 320000
F_IN = 128
HID = 512
NCLS = 16

NC = 2
NS = 16
NW = NC * NS
EPW = 10240
E_PAD = NW * EPW
N_PAD = 10112
RPT = N_PAD // NS
N_PAD_D = 10240
RPT_D = N_PAD_D // NS
BN = N_PAD // 4

@functools.lru_cache(maxsize=None)
def _mesh():
  return plsc.VectorSubcoreMesh(core_axis_name="c", subcore_axis_name="s",
                                num_cores=NC, num_subcores=NS)


@functools.lru_cache(maxsize=None)
def _make_agg(W, G, C, do_scatter=True):
  CH = EPW // C
  NB = G + 1

  def body(u_hbm, src_hbm, dst_hbm, zeros_hbm, out_hbm, *refs):
    src_v, dst_v = refs[0], refs[1]
    bufs = refs[2:2 + NB]
    acc = refs[2 + NB]
    gsems = refs[3 + NB:3 + 2 * NB]
    ssems = refs[3 + 2 * NB:3 + 3 * NB]
    cid = lax.axis_index("c")
    sid = lax.axis_index("s")
    wid = sid * NC + cid
    pltpu.sync_copy(zeros_hbm, acc.at[pl.ds(sid * RPT, RPT)])
    pltpu.sync_copy(src_hbm.at[wid], src_v)
    pltpu.sync_copy(dst_hbm.at[wid], dst_v)
    plsc.subcore_barrier()

    def gather(i):
      b = i % NB
      return pltpu.async_copy(u_hbm.at[src_v.at[i]], bufs[b], gsems[b])

    def scatter(i):
      b = i % NB
      if not do_scatter:
        return None
      return pltpu.async_copy(bufs[b], acc.at[dst_v.at[i]], ssems[b],
                              add=True)

    gp = [None] * CH
    sp = [None] * CH
    for i in range(G):
      gp[i] = gather(i)
    for i in range(CH):
      gp[i].wait()
      sp[i] = scatter(i)
      if i + G < CH:
        if i >= 1 and sp[i - 1] is not None:
          sp[i - 1].wait()
        gp[i + G] = gather(i + G)
    for i in range(max(CH - G - 1, 0), CH):
      if sp[i] is not None:
        sp[i].wait()
    plsc.subcore_barrier()
    pltpu.sync_copy(acc.at[pl.ds(sid * RPT, RPT)],
                    out_hbm.at[cid].at[pl.ds(sid * RPT, RPT)])

  return pl.kernel(
      body,
      out_type=jax.ShapeDtypeStruct((NC, N_PAD, W), jnp.float32),
      mesh=_mesh(),
      compiler_params=pltpu.CompilerParams(use_tc_tiling_on_sc=False),
      scratch_types=[
          pltpu.VMEM((CH, C), jnp.int32),
          pltpu.VMEM((CH, C), jnp.int32),
      ] + [pltpu.VMEM((C, W), jnp.float32)] * NB + [
          pltpu.VMEM_SHARED((N_PAD, W), jnp.float32),
      ] + [pltpu.SemaphoreType.DMA] * (2 * NB),
  )


@functools.lru_cache(maxsize=None)
def _make_deg():
  C = 128
  CH = EPW // C

  def body(dst_hbm, zeros_hbm, out_hbm, dst_v, ones_v, acc, ssem):
    cid = lax.axis_index("c")
    sid = lax.axis_index("s")
    wid = sid * NC + cid
    pltpu.sync_copy(zeros_hbm, acc.at[pl.ds(sid * RPT_D, RPT_D)])
    pltpu.sync_copy(dst_hbm.at[wid], dst_v)
    for j in range(C // 16):
      ones_v[pl.ds(j * 16, 16)] = jnp.full((16,), 1.0, jnp.float32)
    plsc.subcore_barrier()

    def step(j, carry):
      cps = [pltpu.async_copy(ones_v, acc.at[dst_v.at[j * 8 + b]], ssem,
                              add=True) for b in range(8)]
      for cp in cps:
        cp.wait()
      return carry

    lax.fori_loop(0, CH // 8, step, 0)
    plsc.subcore_barrier()
    pltpu.sync_copy(acc.at[pl.ds(sid * RPT_D, RPT_D)],
                    out_hbm.at[cid].at[pl.ds(sid * RPT_D, RPT_D)])

  return pl.kernel(
      body,
      out_type=jax.ShapeDtypeStruct((NC, N_PAD_D), jnp.float32),
      mesh=_mesh(),
      compiler_params=pltpu.CompilerParams(use_tc_tiling_on_sc=False),
      scratch_types=[
          pltpu.VMEM((CH, C), jnp.int32),
          pltpu.VMEM((C,), jnp.float32),
          pltpu.VMEM_SHARED((N_PAD_D,), jnp.float32),
          pltpu.SemaphoreType.DMA,
      ],
  )


def _dis_body(deg_ref, o_ref):
  d = deg_ref[0:1, :N_PAD] + deg_ref[1:2, :N_PAD]
  col = lax.broadcasted_iota(jnp.int32, (1, N_PAD), 1)
  o_ref[...] = jnp.where((col < N) & (d > 0.0),
                         lax.rsqrt(jnp.maximum(d, 1.0)), 0.0)


_dis = pl.pallas_call(
    _dis_body,
    out_shape=jax.ShapeDtypeStruct((1, N_PAD), jnp.float32),
)


def _scale_body(x_ref, dis_ref, o_ref):
  o_ref[...] = x_ref[...] * dis_ref[...]


_scale = pl.pallas_call(
    _scale_body,
    grid=(4,),
    in_specs=[pl.BlockSpec((BN, F_IN), lambda i: (i, 0)),
              pl.BlockSpec((BN, 1), lambda i: (i, 0))],
    out_specs=pl.BlockSpec((BN, F_IN), lambda i: (i, 0)),
    out_shape=jax.ShapeDtypeStruct((N_PAD, F_IN), jnp.float32),
)


def _make_hop(W, alpha, use_P, use_Q, grid=4):
  bn = N_PAD // grid

  def body(*refs):
    i = 0
    agg_ref = refs[i]; i += 1
    dis_ref = refs[i]; i += 1
    p_ref = None
    q_ref = None
    if use_P:
      p_ref = refs[i]; i += 1
    if use_Q:
      q_ref = refs[i]; i += 1
    t_ref = refs[i]; i += 1
    u_ref = refs[i]
    dis = dis_ref[...]
    t = alpha * dis * (agg_ref[0] + agg_ref[1])
    if use_P:
      t = t + p_ref[...]
    if use_Q:
      t = t - q_ref[...]
    t_ref[...] = t
    u_ref[...] = t * dis

  in_specs = [pl.BlockSpec((NC, bn, W), lambda i: (0, i, 0)),
              pl.BlockSpec((bn, 1), lambda i: (i, 0))]
  if use_P:
    in_specs.append(pl.BlockSpec((bn, W), lambda i: (i, 0)))
  if use_Q:
    in_specs.append(pl.BlockSpec((bn, W), lambda i: (i, 0)))
  return pl.pallas_call(
      body,
      grid=(grid,),
      in_specs=in_specs,
      out_specs=[pl.BlockSpec((bn, W), lambda i: (i, 0))] * 2,
      out_shape=[jax.ShapeDtypeStruct((N_PAD, W), jnp.float32)] * 2,
  )


_hop128_first = _make_hop(F_IN, -1.0, False, False)
_hop128 = _make_hop(F_IN, -2.0, False, True)
_hop16_p = _make_hop(NCLS, -2.0, True, False, grid=1)
_hop16_pq = _make_hop(NCLS, -2.0, True, True, grid=1)


def _elu(v):
  return jnp.where(v > 0.0, v, jnp.exp(jnp.minimum(v, 0.0)) - 1.0)


def _mm_body(t0, t1, t2, t3, t4, w1, b1r, w2, dis_ref,
             z0o, z1o, z2o, z3o, z4o, ubo):
  h = jnp.dot(t0[...], w1[0], preferred_element_type=jnp.float32)
  h = h + jnp.dot(t1[...], w1[1], preferred_element_type=jnp.float32)
  h = h + jnp.dot(t2[...], w1[2], preferred_element_type=jnp.float32)
  h = h + jnp.dot(t3[...], w1[3], preferred_element_type=jnp.float32)
  h = h + jnp.dot(t4[...], w1[4], preferred_element_type=jnp.float32)
  h = _elu(h + b1r[...])
  z4 = jnp.dot(h, w2[4], preferred_element_type=jnp.float32)
  z0o[...] = jnp.dot(h, w2[0], preferred_element_type=jnp.float32)
  z1o[...] = jnp.dot(h, w2[1], preferred_element_type=jnp.float32)
  z2o[...] = jnp.dot(h, w2[2], preferred_element_type=jnp.float32)
  z3o[...] = jnp.dot(h, w2[3], preferred_element_type=jnp.float32)
  z4o[...] = z4
  ubo[...] = z4 * dis_ref[...]


_mm = pl.pallas_call(
    _mm_body,
    grid=(4,),
    in_specs=[pl.BlockSpec((BN, F_IN), lambda i: (i, 0))] * 5 + [
        pl.BlockSpec((5, F_IN, HID), lambda i: (0, 0, 0)),
        pl.BlockSpec((1, HID), lambda i: (0, 0)),
        pl.BlockSpec((5, HID, NCLS), lambda i: (0, 0, 0)),
        pl.BlockSpec((BN, 1), lambda i: (i, 0)),
    ],
    out_specs=[pl.BlockSpec((BN, NCLS), lambda i: (i, 0))] * 6,
    out_shape=[jax.ShapeDtypeStruct((N_PAD, NCLS), jnp.float32)] * 6,
)


def _final_body(z0_ref, agg_ref, dis_ref, b2_ref, q_ref, o_ref):
  o = (z0_ref[...] - dis_ref[...] * (agg_ref[0] + agg_ref[1])
       - q_ref[...] + b2_ref[...])
  o = _elu(o)
  m = jnp.max(o, axis=1, keepdims=True)
  e = jnp.exp(o - m)
  s = jnp.sum(e, axis=1, keepdims=True)
  o_ref[...] = o - m - jnp.log(s)


_final = pl.pallas_call(
    _final_body,
    grid=(1,),
    in_specs=[pl.BlockSpec((N, NCLS), lambda i: (0, 0)),
              pl.BlockSpec((NC, N, NCLS), lambda i: (0, 0, 0)),
              pl.BlockSpec((N, 1), lambda i: (0, 0)),
              pl.BlockSpec((1, NCLS), lambda i: (0, 0)),
              pl.BlockSpec((N, NCLS), lambda i: (0, 0))],
    out_specs=pl.BlockSpec((N, NCLS), lambda i: (0, 0)),
    out_shape=jax.ShapeDtypeStruct((N, NCLS), jnp.float32),
)


def kernel(x, edge_index, W1, b1, W2, b2):
  src = edge_index[0]
  dst = edge_index[1]
  pad = E_PAD - E
  padv = N + (jnp.arange(pad, dtype=jnp.int32) % (N_PAD - N))
  src_flat = jnp.concatenate([src, padv]).reshape(NW, EPW)
  dst_flat = jnp.concatenate([dst, padv]).reshape(NW, EPW)
  srcp64 = src_flat.reshape(NW, EPW // 64, 64)
  dstp64 = dst_flat.reshape(NW, EPW // 64, 64)
  srcp = src_flat.reshape(NW, EPW // 128, 128)
  dstp = dst_flat.reshape(NW, EPW // 128, 128)
  x_pad = jnp.concatenate(
      [x, jnp.zeros((N_PAD - N, F_IN), jnp.float32)], axis=0)
  z128 = jnp.zeros((RPT, F_IN), jnp.float32)
  z16 = jnp.zeros((RPT, NCLS), jnp.float32)
  zdeg = jnp.zeros((RPT_D,), jnp.float32)

  degp = _make_deg()(dstp, zdeg)
  dis = _dis(degp).reshape(N_PAD, 1)
  u = _scale(x_pad, dis)

  agg128 = _make_agg(F_IN, 2, 64, False)
  agg16 = _make_agg(NCLS, 6, 128)
  agg = agg128(u, srcp64, dstp64, z128)
  tx1, u = _hop128_first(agg, dis)
  agg = agg128(u, srcp64, dstp64, z128)
  tx2, u = _hop128(agg, dis, x_pad)
  agg = agg128(u, srcp64, dstp64, z128)
  tx3, u = _hop128(agg, dis, tx1)
  agg = agg128(u, srcp64, dstp64, z128)
  tx4, _ = _hop128(agg, dis, tx2)

  z0, z1, z2, z3, z4, ub = _mm(
      x_pad, tx1, tx2, tx3, tx4, W1, b1.reshape(1, HID), W2, dis)

  agg = agg16(ub, srcp, dstp, z16)
  b3, ub = _hop16_p(agg, dis, z3)
  agg = agg16(ub, srcp, dstp, z16)
  bb2, ub = _hop16_pq(agg, dis, z2, z4)
  agg = agg16(ub, srcp, dstp, z16)
  b1_, ub = _hop16_pq(agg, dis, z1, b3)
  agg = agg16(ub, srcp, dstp, z16)
  return _final(z0, agg, dis, b2.reshape(1, NCLS), bb2)


# speedup vs baseline: 1.1624x; 1.1624x over previous
import functools

import jax
import jax.numpy as jnp
from jax import lax
from jax.experimental import pallas as pl
from jax.experimental.pallas import tpu as pltpu
from jax.experimental.pallas import tpu_sc as plsc

N = 10000
E = 320000
F_IN = 128
HID = 512
NCLS = 16

NC = 2
NS = 16
NW = NC * NS
EPW = 10240
E_PAD = NW * EPW
N_PAD = 10112
RPT = N_PAD // NS
N_PAD_D = 10240
RPT_D = N_PAD_D // NS
BN = N_PAD // 4

@functools.lru_cache(maxsize=None)
def _mesh():
  return plsc.VectorSubcoreMesh(core_axis_name="c", subcore_axis_name="s",
                                num_cores=NC, num_subcores=NS)


@functools.lru_cache(maxsize=None)
def _make_agg(W, G, C, dt=jnp.float32):
  CH = EPW // C
  NB = G + 1

  def body(u_hbm, src_hbm, dst_hbm, zeros_hbm, out_hbm, *refs):
    src_v, dst_v = refs[0], refs[1]
    bufs = refs[2:2 + NB]
    acc = refs[2 + NB]
    gsems = refs[3 + NB:3 + 2 * NB]
    ssems = refs[3 + 2 * NB:3 + 3 * NB]
    cid = lax.axis_index("c")
    sid = lax.axis_index("s")
    wid = sid * NC + cid
    pltpu.sync_copy(zeros_hbm, acc.at[pl.ds(sid * RPT, RPT)])
    pltpu.sync_copy(src_hbm.at[wid], src_v)
    pltpu.sync_copy(dst_hbm.at[wid], dst_v)
    plsc.subcore_barrier()

    def gather(i):
      b = i % NB
      return pltpu.async_copy(u_hbm.at[src_v.at[i]], bufs[b], gsems[b])

    def scatter(i):
      b = i % NB
      return pltpu.async_copy(bufs[b], acc.at[dst_v.at[i]], ssems[b],
                              add=True)

    gp = [None] * CH
    sp = [None] * CH
    for i in range(G):
      gp[i] = gather(i)
    for i in range(CH):
      gp[i].wait()
      sp[i] = scatter(i)
      if i + G < CH:
        if i >= 1:
          sp[i - 1].wait()
        gp[i + G] = gather(i + G)
    for i in range(max(CH - G - 1, 0), CH):
      if sp[i] is not None:
        sp[i].wait()
    plsc.subcore_barrier()
    pltpu.sync_copy(acc.at[pl.ds(sid * RPT, RPT)],
                    out_hbm.at[cid].at[pl.ds(sid * RPT, RPT)])

  return pl.kernel(
      body,
      out_type=jax.ShapeDtypeStruct((NC, N_PAD, W), dt),
      mesh=_mesh(),
      compiler_params=pltpu.CompilerParams(use_tc_tiling_on_sc=False),
      scratch_types=[
          pltpu.VMEM((CH, C), jnp.int32),
          pltpu.VMEM((CH, C), jnp.int32),
      ] + [pltpu.VMEM((C, W), dt)] * NB + [
          pltpu.VMEM_SHARED((N_PAD, W), dt),
      ] + [pltpu.SemaphoreType.DMA] * (2 * NB),
  )


@functools.lru_cache(maxsize=None)
def _make_deg():
  C = 128
  CH = EPW // C

  def body(dst_hbm, zeros_hbm, out_hbm, dst_v, ones_v, acc, ssem):
    cid = lax.axis_index("c")
    sid = lax.axis_index("s")
    wid = sid * NC + cid
    pltpu.sync_copy(zeros_hbm, acc.at[pl.ds(sid * RPT_D, RPT_D)])
    pltpu.sync_copy(dst_hbm.at[wid], dst_v)
    for j in range(C // 16):
      ones_v[pl.ds(j * 16, 16)] = jnp.full((16,), 1.0, jnp.float32)
    plsc.subcore_barrier()

    def step(j, carry):
      cps = [pltpu.async_copy(ones_v, acc.at[dst_v.at[j * 8 + b]], ssem,
                              add=True) for b in range(8)]
      for cp in cps:
        cp.wait()
      return carry

    lax.fori_loop(0, CH // 8, step, 0)
    plsc.subcore_barrier()
    pltpu.sync_copy(acc.at[pl.ds(sid * RPT_D, RPT_D)],
                    out_hbm.at[cid].at[pl.ds(sid * RPT_D, RPT_D)])

  return pl.kernel(
      body,
      out_type=jax.ShapeDtypeStruct((NC, N_PAD_D), jnp.float32),
      mesh=_mesh(),
      compiler_params=pltpu.CompilerParams(use_tc_tiling_on_sc=False),
      scratch_types=[
          pltpu.VMEM((CH, C), jnp.int32),
          pltpu.VMEM((C,), jnp.float32),
          pltpu.VMEM_SHARED((N_PAD_D,), jnp.float32),
          pltpu.SemaphoreType.DMA,
      ],
  )


def _dis_body(deg_ref, o_ref):
  d = deg_ref[0:1, :N_PAD] + deg_ref[1:2, :N_PAD]
  col = lax.broadcasted_iota(jnp.int32, (1, N_PAD), 1)
  o_ref[...] = jnp.where((col < N) & (d > 0.0),
                         lax.rsqrt(jnp.maximum(d, 1.0)), 0.0)


_dis = pl.pallas_call(
    _dis_body,
    out_shape=jax.ShapeDtypeStruct((1, N_PAD), jnp.float32),
)


def _scale_body(x_ref, dis_ref, o_ref):
  o_ref[...] = (x_ref[...] * dis_ref[...]).astype(jnp.bfloat16)


_scale = pl.pallas_call(
    _scale_body,
    grid=(4,),
    in_specs=[pl.BlockSpec((BN, F_IN), lambda i: (i, 0)),
              pl.BlockSpec((BN, 1), lambda i: (i, 0))],
    out_specs=pl.BlockSpec((BN, F_IN), lambda i: (i, 0)),
    out_shape=jax.ShapeDtypeStruct((N_PAD, F_IN), jnp.bfloat16),
)


def _make_hop(W, alpha, use_P, use_Q, grid=4, dt=jnp.float32):
  bn = N_PAD // grid

  def body(*refs):
    i = 0
    agg_ref = refs[i]; i += 1
    dis_ref = refs[i]; i += 1
    p_ref = None
    q_ref = None
    if use_P:
      p_ref = refs[i]; i += 1
    if use_Q:
      q_ref = refs[i]; i += 1
    t_ref = refs[i]; i += 1
    u_ref = refs[i]
    dis = dis_ref[...]
    a = (agg_ref[0].astype(jnp.float32) + agg_ref[1].astype(jnp.float32))
    t = alpha * dis * a
    if use_P:
      t = t + p_ref[...]
    if use_Q:
      t = t - q_ref[...]
    t_ref[...] = t
    u_ref[...] = (t * dis).astype(dt)

  in_specs = [pl.BlockSpec((NC, bn, W), lambda i: (0, i, 0)),
              pl.BlockSpec((bn, 1), lambda i: (i, 0))]
  if use_P:
    in_specs.append(pl.BlockSpec((bn, W), lambda i: (i, 0)))
  if use_Q:
    in_specs.append(pl.BlockSpec((bn, W), lambda i: (i, 0)))
  return pl.pallas_call(
      body,
      grid=(grid,),
      in_specs=in_specs,
      out_specs=[pl.BlockSpec((bn, W), lambda i: (i, 0))] * 2,
      out_shape=[jax.ShapeDtypeStruct((N_PAD, W), jnp.float32),
                 jax.ShapeDtypeStruct((N_PAD, W), dt)],
  )


_hop128_first = _make_hop(F_IN, -1.0, False, False, dt=jnp.bfloat16)
_hop128 = _make_hop(F_IN, -2.0, False, True, dt=jnp.bfloat16)
_hop16_p = _make_hop(NCLS, -2.0, True, False, grid=1)
_hop16_pq = _make_hop(NCLS, -2.0, True, True, grid=1)


def _elu(v):
  return jnp.where(v > 0.0, v, jnp.exp(jnp.minimum(v, 0.0)) - 1.0)


def _mm_body(t0, t1, t2, t3, t4, w1, b1r, w2, dis_ref,
             z0o, z1o, z2o, z3o, z4o, ubo):
  h = jnp.dot(t0[...], w1[0], preferred_element_type=jnp.float32)
  h = h + jnp.dot(t1[...], w1[1], preferred_element_type=jnp.float32)
  h = h + jnp.dot(t2[...], w1[2], preferred_element_type=jnp.float32)
  h = h + jnp.dot(t3[...], w1[3], preferred_element_type=jnp.float32)
  h = h + jnp.dot(t4[...], w1[4], preferred_element_type=jnp.float32)
  h = _elu(h + b1r[...])
  z4 = jnp.dot(h, w2[4], preferred_element_type=jnp.float32)
  z0o[...] = jnp.dot(h, w2[0], preferred_element_type=jnp.float32)
  z1o[...] = jnp.dot(h, w2[1], preferred_element_type=jnp.float32)
  z2o[...] = jnp.dot(h, w2[2], preferred_element_type=jnp.float32)
  z3o[...] = jnp.dot(h, w2[3], preferred_element_type=jnp.float32)
  z4o[...] = z4
  ubo[...] = z4 * dis_ref[...]


_mm = pl.pallas_call(
    _mm_body,
    grid=(4,),
    in_specs=[pl.BlockSpec((BN, F_IN), lambda i: (i, 0))] * 5 + [
        pl.BlockSpec((5, F_IN, HID), lambda i: (0, 0, 0)),
        pl.BlockSpec((1, HID), lambda i: (0, 0)),
        pl.BlockSpec((5, HID, NCLS), lambda i: (0, 0, 0)),
        pl.BlockSpec((BN, 1), lambda i: (i, 0)),
    ],
    out_specs=[pl.BlockSpec((BN, NCLS), lambda i: (i, 0))] * 6,
    out_shape=[jax.ShapeDtypeStruct((N_PAD, NCLS), jnp.float32)] * 6,
)


def _final_body(z0_ref, agg_ref, dis_ref, b2_ref, q_ref, o_ref):
  o = (z0_ref[...] - dis_ref[...] * (agg_ref[0] + agg_ref[1])
       - q_ref[...] + b2_ref[...])
  o = _elu(o)
  m = jnp.max(o, axis=1, keepdims=True)
  e = jnp.exp(o - m)
  s = jnp.sum(e, axis=1, keepdims=True)
  o_ref[...] = o - m - jnp.log(s)


_final = pl.pallas_call(
    _final_body,
    grid=(1,),
    in_specs=[pl.BlockSpec((N, NCLS), lambda i: (0, 0)),
              pl.BlockSpec((NC, N, NCLS), lambda i: (0, 0, 0)),
              pl.BlockSpec((N, 1), lambda i: (0, 0)),
              pl.BlockSpec((1, NCLS), lambda i: (0, 0)),
              pl.BlockSpec((N, NCLS), lambda i: (0, 0))],
    out_specs=pl.BlockSpec((N, NCLS), lambda i: (0, 0)),
    out_shape=jax.ShapeDtypeStruct((N, NCLS), jnp.float32),
)


def kernel(x, edge_index, W1, b1, W2, b2):
  src = edge_index[0]
  dst = edge_index[1]
  pad = E_PAD - E
  padv = N + (jnp.arange(pad, dtype=jnp.int32) % (N_PAD - N))
  src_flat = jnp.concatenate([src, padv]).reshape(NW, EPW)
  dst_flat = jnp.concatenate([dst, padv]).reshape(NW, EPW)
  srcp = src_flat.reshape(NW, EPW // 128, 128)
  dstp = dst_flat.reshape(NW, EPW // 128, 128)
  x_pad = jnp.concatenate(
      [x, jnp.zeros((N_PAD - N, F_IN), jnp.float32)], axis=0)
  z128 = jnp.zeros((RPT, F_IN), jnp.bfloat16)
  z16 = jnp.zeros((RPT, NCLS), jnp.float32)
  zdeg = jnp.zeros((RPT_D,), jnp.float32)

  degp = _make_deg()(dstp, zdeg)
  dis = _dis(degp).reshape(N_PAD, 1)
  u = _scale(x_pad, dis)

  agg128 = _make_agg(F_IN, 6, 128, jnp.bfloat16)
  agg16 = _make_agg(NCLS, 6, 128)
  agg = agg128(u, srcp, dstp, z128)
  tx1, u = _hop128_first(agg, dis)
  agg = agg128(u, srcp, dstp, z128)
  tx2, u = _hop128(agg, dis, x_pad)
  agg = agg128(u, srcp, dstp, z128)
  tx3, u = _hop128(agg, dis, tx1)
  agg = agg128(u, srcp, dstp, z128)
  tx4, _ = _hop128(agg, dis, tx2)

  z0, z1, z2, z3, z4, ub = _mm(
      x_pad, tx1, tx2, tx3, tx4, W1, b1.reshape(1, HID), W2, dis)

  agg = agg16(ub, srcp, dstp, z16)
  b3, ub = _hop16_p(agg, dis, z3)
  agg = agg16(ub, srcp, dstp, z16)
  bb2, ub = _hop16_pq(agg, dis, z2, z4)
  agg = agg16(ub, srcp, dstp, z16)
  b1_, ub = _hop16_pq(agg, dis, z1, b3)
  agg = agg16(ub, srcp, dstp, z16)
  return _final(z0, agg, dis, b2.reshape(1, NCLS), bb2)


# speedup vs baseline: 1.1643x; 1.0016x over previous
import functools

import jax
import jax.numpy as jnp
from jax import lax
from jax.experimental import pallas as pl
from jax.experimental.pallas import tpu as pltpu
from jax.experimental.pallas import tpu_sc as plsc

N = 10000
E = 320000
F_IN = 128
HID = 512
NCLS = 16

NC = 2
NS = 16
NW = NC * NS
EPW = 10240
E_PAD = NW * EPW
N_PAD = 10112
RPT = N_PAD // NS
N_PAD_D = 10240
RPT_D = N_PAD_D // NS
BN = N_PAD // 4

@functools.lru_cache(maxsize=None)
def _mesh():
  return plsc.VectorSubcoreMesh(core_axis_name="c", subcore_axis_name="s",
                                num_cores=NC, num_subcores=NS)


@functools.lru_cache(maxsize=None)
def _make_agg(W, G, C, dt=jnp.float32):
  CH = EPW // C
  NB = G + 1

  def body(u_hbm, src_hbm, dst_hbm, zeros_hbm, out_hbm, *refs):
    src_v, dst_v = refs[0], refs[1]
    bufs = refs[2:2 + NB]
    acc = refs[2 + NB]
    gsems = refs[3 + NB:3 + 2 * NB]
    ssems = refs[3 + 2 * NB:3 + 3 * NB]
    cid = lax.axis_index("c")
    sid = lax.axis_index("s")
    wid = sid * NC + cid
    pltpu.sync_copy(zeros_hbm, acc.at[pl.ds(sid * RPT, RPT)])
    pltpu.sync_copy(src_hbm.at[wid], src_v)
    pltpu.sync_copy(dst_hbm.at[wid], dst_v)
    plsc.subcore_barrier()

    def gather(i):
      b = i % NB
      return pltpu.async_copy(u_hbm.at[src_v.at[i]], bufs[b], gsems[b])

    def scatter(i):
      b = i % NB
      return pltpu.async_copy(bufs[b], acc.at[dst_v.at[i]], ssems[b],
                              add=True)

    gp = [None] * CH
    sp = [None] * CH
    for i in range(G):
      gp[i] = gather(i)
    for i in range(CH):
      gp[i].wait()
      sp[i] = scatter(i)
      if i + G < CH:
        if i >= 1:
          sp[i - 1].wait()
        gp[i + G] = gather(i + G)
    for i in range(max(CH - G - 1, 0), CH):
      if sp[i] is not None:
        sp[i].wait()
    plsc.subcore_barrier()
    pltpu.sync_copy(acc.at[pl.ds(sid * RPT, RPT)],
                    out_hbm.at[cid].at[pl.ds(sid * RPT, RPT)])

  return pl.kernel(
      body,
      out_type=jax.ShapeDtypeStruct((NC, N_PAD, W), dt),
      mesh=_mesh(),
      compiler_params=pltpu.CompilerParams(use_tc_tiling_on_sc=False),
      scratch_types=[
          pltpu.VMEM((CH, C), jnp.int32),
          pltpu.VMEM((CH, C), jnp.int32),
      ] + [pltpu.VMEM((C, W), dt)] * NB + [
          pltpu.VMEM_SHARED((N_PAD, W), dt),
      ] + [pltpu.SemaphoreType.DMA] * (2 * NB),
  )


@functools.lru_cache(maxsize=None)
def _make_deg():
  C = 128
  CH = EPW // C

  def body(dst_hbm, zeros_hbm, out_hbm, dst_v, ones_v, acc, ssem):
    cid = lax.axis_index("c")
    sid = lax.axis_index("s")
    wid = sid * NC + cid
    pltpu.sync_copy(zeros_hbm, acc.at[pl.ds(sid * RPT_D, RPT_D)])
    pltpu.sync_copy(dst_hbm.at[wid], dst_v)
    for j in range(C // 16):
      ones_v[pl.ds(j * 16, 16)] = jnp.full((16,), 1.0, jnp.float32)
    plsc.subcore_barrier()

    def step(j, carry):
      cps = [pltpu.async_copy(ones_v, acc.at[dst_v.at[j * 8 + b]], ssem,
                              add=True) for b in range(8)]
      for cp in cps:
        cp.wait()
      return carry

    lax.fori_loop(0, CH // 8, step, 0)
    plsc.subcore_barrier()
    pltpu.sync_copy(acc.at[pl.ds(sid * RPT_D, RPT_D)],
                    out_hbm.at[cid].at[pl.ds(sid * RPT_D, RPT_D)])

  return pl.kernel(
      body,
      out_type=jax.ShapeDtypeStruct((NC, N_PAD_D), jnp.float32),
      mesh=_mesh(),
      compiler_params=pltpu.CompilerParams(use_tc_tiling_on_sc=False),
      scratch_types=[
          pltpu.VMEM((CH, C), jnp.int32),
          pltpu.VMEM((C,), jnp.float32),
          pltpu.VMEM_SHARED((N_PAD_D,), jnp.float32),
          pltpu.SemaphoreType.DMA,
      ],
  )


def _dis_body(deg_ref, o_ref):
  d = deg_ref[0:1, :N_PAD] + deg_ref[1:2, :N_PAD]
  col = lax.broadcasted_iota(jnp.int32, (1, N_PAD), 1)
  o_ref[...] = jnp.where((col < N) & (d > 0.0),
                         lax.rsqrt(jnp.maximum(d, 1.0)), 0.0)


_dis = pl.pallas_call(
    _dis_body,
    out_shape=jax.ShapeDtypeStruct((1, N_PAD), jnp.float32),
)


def _scale_body(x_ref, dis_ref, o_ref):
  o_ref[...] = (x_ref[...] * dis_ref[...]).astype(jnp.bfloat16)


_scale = pl.pallas_call(
    _scale_body,
    grid=(4,),
    in_specs=[pl.BlockSpec((BN, F_IN), lambda i: (i, 0)),
              pl.BlockSpec((BN, 1), lambda i: (i, 0))],
    out_specs=pl.BlockSpec((BN, F_IN), lambda i: (i, 0)),
    out_shape=jax.ShapeDtypeStruct((N_PAD, F_IN), jnp.bfloat16),
)


def _make_hop(W, alpha, use_P, use_Q, grid=4, dt=jnp.float32):
  bn = N_PAD // grid

  def body(*refs):
    i = 0
    agg_ref = refs[i]; i += 1
    dis_ref = refs[i]; i += 1
    p_ref = None
    q_ref = None
    if use_P:
      p_ref = refs[i]; i += 1
    if use_Q:
      q_ref = refs[i]; i += 1
    t_ref = refs[i]; i += 1
    u_ref = refs[i]
    dis = dis_ref[...]
    a = (agg_ref[0].astype(jnp.float32) + agg_ref[1].astype(jnp.float32))
    t = alpha * dis * a
    if use_P:
      t = t + p_ref[...]
    if use_Q:
      t = t - q_ref[...]
    t_ref[...] = t
    u_ref[...] = (t * dis).astype(dt)

  in_specs = [pl.BlockSpec((NC, bn, W), lambda i: (0, i, 0)),
              pl.BlockSpec((bn, 1), lambda i: (i, 0))]
  if use_P:
    in_specs.append(pl.BlockSpec((bn, W), lambda i: (i, 0)))
  if use_Q:
    in_specs.append(pl.BlockSpec((bn, W), lambda i: (i, 0)))
  return pl.pallas_call(
      body,
      grid=(grid,),
      in_specs=in_specs,
      out_specs=[pl.BlockSpec((bn, W), lambda i: (i, 0))] * 2,
      out_shape=[jax.ShapeDtypeStruct((N_PAD, W), jnp.float32),
                 jax.ShapeDtypeStruct((N_PAD, W), dt)],
  )


_hop128_first = _make_hop(F_IN, -1.0, False, False, dt=jnp.bfloat16)
_hop128 = _make_hop(F_IN, -2.0, False, True, dt=jnp.bfloat16)
_hop16_p = _make_hop(NCLS, -2.0, True, False, grid=1)
_hop16_pq = _make_hop(NCLS, -2.0, True, True, grid=1)


def _elu(v):
  return jnp.where(v > 0.0, v, jnp.exp(jnp.minimum(v, 0.0)) - 1.0)


def _mm_body(t0, t1, t2, t3, t4, w1, b1r, w2, dis_ref,
             z0o, z1o, z2o, z3o, z4o, ubo):
  h = jnp.dot(t0[...], w1[0], preferred_element_type=jnp.float32)
  h = h + jnp.dot(t1[...], w1[1], preferred_element_type=jnp.float32)
  h = h + jnp.dot(t2[...], w1[2], preferred_element_type=jnp.float32)
  h = h + jnp.dot(t3[...], w1[3], preferred_element_type=jnp.float32)
  h = h + jnp.dot(t4[...], w1[4], preferred_element_type=jnp.float32)
  h = _elu(h + b1r[...])
  z4 = jnp.dot(h, w2[4], preferred_element_type=jnp.float32)
  z0o[...] = jnp.dot(h, w2[0], preferred_element_type=jnp.float32)
  z1o[...] = jnp.dot(h, w2[1], preferred_element_type=jnp.float32)
  z2o[...] = jnp.dot(h, w2[2], preferred_element_type=jnp.float32)
  z3o[...] = jnp.dot(h, w2[3], preferred_element_type=jnp.float32)
  z4o[...] = z4
  ubo[...] = z4 * dis_ref[...]


_mm = pl.pallas_call(
    _mm_body,
    grid=(4,),
    in_specs=[pl.BlockSpec((BN, F_IN), lambda i: (i, 0))] * 5 + [
        pl.BlockSpec((5, F_IN, HID), lambda i: (0, 0, 0)),
        pl.BlockSpec((1, HID), lambda i: (0, 0)),
        pl.BlockSpec((5, HID, NCLS), lambda i: (0, 0, 0)),
        pl.BlockSpec((BN, 1), lambda i: (i, 0)),
    ],
    out_specs=[pl.BlockSpec((BN, NCLS), lambda i: (i, 0))] * 6,
    out_shape=[jax.ShapeDtypeStruct((N_PAD, NCLS), jnp.float32)] * 6,
)


def _final_body(z0_ref, agg_ref, dis_ref, b2_ref, q_ref, o_ref):
  o = (z0_ref[...] - dis_ref[...] * (agg_ref[0] + agg_ref[1])
       - q_ref[...] + b2_ref[...])
  o = _elu(o)
  m = jnp.max(o, axis=1, keepdims=True)
  e = jnp.exp(o - m)
  s = jnp.sum(e, axis=1, keepdims=True)
  o_ref[...] = o - m - jnp.log(s)


_final = pl.pallas_call(
    _final_body,
    grid=(1,),
    in_specs=[pl.BlockSpec((N, NCLS), lambda i: (0, 0)),
              pl.BlockSpec((NC, N, NCLS), lambda i: (0, 0, 0)),
              pl.BlockSpec((N, 1), lambda i: (0, 0)),
              pl.BlockSpec((1, NCLS), lambda i: (0, 0)),
              pl.BlockSpec((N, NCLS), lambda i: (0, 0))],
    out_specs=pl.BlockSpec((N, NCLS), lambda i: (0, 0)),
    out_shape=jax.ShapeDtypeStruct((N, NCLS), jnp.float32),
)


def kernel(x, edge_index, W1, b1, W2, b2):
  src = edge_index[0]
  dst = edge_index[1]
  pad = E_PAD - E
  padv = N + (jnp.arange(pad, dtype=jnp.int32) % (N_PAD - N))
  src_flat = jnp.concatenate([src, padv]).reshape(NW, EPW)
  dst_flat = jnp.concatenate([dst, padv]).reshape(NW, EPW)
  srcp = src_flat.reshape(NW, EPW // 128, 128)
  dstp = dst_flat.reshape(NW, EPW // 128, 128)
  x_pad = jnp.concatenate(
      [x, jnp.zeros((N_PAD - N, F_IN), jnp.float32)], axis=0)
  z128 = jnp.zeros((RPT, F_IN), jnp.bfloat16)
  z16 = jnp.zeros((RPT, NCLS), jnp.float32)
  zdeg = jnp.zeros((RPT_D,), jnp.float32)

  degp = _make_deg()(dstp, zdeg)
  dis = _dis(degp).reshape(N_PAD, 1)
  u = _scale(x_pad, dis)

  agg128 = _make_agg(F_IN, 6, 128, jnp.bfloat16)
  agg16 = _make_agg(NCLS, 12, 128)
  agg = agg128(u, srcp, dstp, z128)
  tx1, u = _hop128_first(agg, dis)
  agg = agg128(u, srcp, dstp, z128)
  tx2, u = _hop128(agg, dis, x_pad)
  agg = agg128(u, srcp, dstp, z128)
  tx3, u = _hop128(agg, dis, tx1)
  agg = agg128(u, srcp, dstp, z128)
  tx4, _ = _hop128(agg, dis, tx2)

  z0, z1, z2, z3, z4, ub = _mm(
      x_pad, tx1, tx2, tx3, tx4, W1, b1.reshape(1, HID), W2, dis)

  agg = agg16(ub, srcp, dstp, z16)
  b3, ub = _hop16_p(agg, dis, z3)
  agg = agg16(ub, srcp, dstp, z16)
  bb2, ub = _hop16_pq(agg, dis, z2, z4)
  agg = agg16(ub, srcp, dstp, z16)
  b1_, ub = _hop16_pq(agg, dis, z1, b3)
  agg = agg16(ub, srcp, dstp, z16)
  return _final(z0, agg, dis, b2.reshape(1, NCLS), bb2)
